# baseline (speedup 1.0000x reference)
import functools
import jax
import jax.numpy as jnp
from jax import lax
from jax.experimental import pallas as pl
from jax.experimental.pallas import tpu as pltpu
from jax.experimental.pallas import tpu_sc as plsc

_N = 8192
_NC = 2
_NS = 16
_NW = _NC * _NS
_QW = _N // _NW
_L = 16
_QU = 8


def _bf16r(x):
    u = lax.bitcast_convert_type(x, jnp.uint32)
    r = (u + jnp.uint32(0x7FFF) + ((u >> 16) & jnp.uint32(1))) & jnp.uint32(0xFFFF0000)
    return lax.bitcast_convert_type(r, jnp.float32)


def _fill_prep(kx_v, ky_v, kz_v, kxr_v, kyr_v, kzr_v, ksq_v):
    def body(kb, carry):
        o = kb * _L
        kx = kx_v[pl.ds(o, _L)]
        ky = ky_v[pl.ds(o, _L)]
        kz = kz_v[pl.ds(o, _L)]
        ksq_v[pl.ds(o, _L)] = kx * kx + ky * ky + kz * kz
        kxr_v[pl.ds(o, _L)] = _bf16r(kx)
        kyr_v[pl.ds(o, _L)] = _bf16r(ky)
        kzr_v[pl.ds(o, _L)] = _bf16r(kz)
        return carry
    lax.fori_loop(0, _N // _L, body, jnp.float32(0.0))


def _direction_sum(qsq_v, qxr_v, qyr_v, qzr_v,
                   kxr_v, kyr_v, kzr_v, ksq_v, qbase):
    inf16 = jnp.full((_L,), 3.0e38, jnp.float32)

    def qgroup_body(g, total):
        qb = qbase + g * _QU
        qxv = qxr_v[pl.ds(qb, _L)]
        qyv = qyr_v[pl.ds(qb, _L)]
        qzv = qzr_v[pl.ds(qb, _L)]
        qsqv = qsq_v[pl.ds(qb, _L)]
        cxv = qxv * (-2.0)
        cyv = qyv * (-2.0)
        czv = qzv * (-2.0)
        cs = [(cxv[j], cyv[j], czv[j]) for j in range(_QU)]

        @plsc.parallel_loop(0, _N, _L, unroll=2, carry=(inf16,) * _QU)
        def mins(o, mins_c):
            kx = kxr_v[pl.ds(o, _L)]
            ky = kyr_v[pl.ds(o, _L)]
            kz = kzr_v[pl.ds(o, _L)]
            ksq = ksq_v[pl.ds(o, _L)]
            out = []
            for j in range(_QU):
                d2 = (ksq + kx * cs[j][0]) + (ky * cs[j][1] + kz * cs[j][2])
                out.append(jnp.minimum(mins_c[j], d2))
            return tuple(out)

        for j in range(_QU):
            srt, _ = plsc.sort_key_val(mins[j], mins[j])
            total = total + jnp.maximum(srt[0] + qsqv[j], 0.0)
        return total

    return lax.fori_loop(0, _QW // _QU, qgroup_body, jnp.float32(0.0))


def _chamfer_sc(gtx, gty, gtz, gnx, gny, gnz, out,
                gtx_v, gty_v, gtz_v, gnx_v, gny_v, gnz_v,
                gtxr_v, gtyr_v, gtzr_v, gnxr_v, gnyr_v, gnzr_v,
                gtsq_v, gnsq_v, res_v):
    w = lax.axis_index("s") * _NC + lax.axis_index("c")
    pltpu.sync_copy(gtx, gtx_v)
    pltpu.sync_copy(gty, gty_v)
    pltpu.sync_copy(gtz, gtz_v)
    pltpu.sync_copy(gnx, gnx_v)
    pltpu.sync_copy(gny, gny_v)
    pltpu.sync_copy(gnz, gnz_v)
    _fill_prep(gtx_v, gty_v, gtz_v, gtxr_v, gtyr_v, gtzr_v, gtsq_v)
    _fill_prep(gnx_v, gny_v, gnz_v, gnxr_v, gnyr_v, gnzr_v, gnsq_v)
    qbase = w * _QW
    s1 = _direction_sum(gtsq_v, gtxr_v, gtyr_v, gtzr_v,
                        gnxr_v, gnyr_v, gnzr_v, gnsq_v, qbase)
    s2 = _direction_sum(gnsq_v, gnxr_v, gnyr_v, gnzr_v,
                        gtxr_v, gtyr_v, gtzr_v, gtsq_v, qbase)
    res_v[...] = jnp.full((_L,), (s1 + s2) * (1.0 / _L), jnp.float32)
    pltpu.sync_copy(res_v, out.at[pl.ds(w * _L, _L)])


_sc_call = functools.partial(
    pl.kernel,
    out_type=jax.ShapeDtypeStruct((_NW * _L,), jnp.float32),
    mesh=plsc.VectorSubcoreMesh(core_axis_name="c", subcore_axis_name="s"),
    scratch_types=[pltpu.VMEM((_N,), jnp.float32)] * 14
    + [pltpu.VMEM((_L,), jnp.float32)],
    compiler_params=pltpu.CompilerParams(needs_layout_passes=False),
)(_chamfer_sc)


def kernel(gt_points, gen_points):
    partial = _sc_call(
        gt_points[:, 0], gt_points[:, 1], gt_points[:, 2],
        gen_points[:, 0], gen_points[:, 1], gen_points[:, 2],
    )
    return jnp.sum(partial) * (1.0 / _N)


# speedup vs baseline: 8.1034x; 8.1034x over previous
import jax
import jax.numpy as jnp
from jax.experimental import pallas as pl
from jax.experimental.pallas import tpu as pltpu

_N = 8192
_B = 1024
_NB = _N // _B


def _chamfer_body(gt_ref, gent_ref, out_ref, colmin_ref):
    i = pl.program_id(0)
    gtb = gt_ref[...]
    genb = gent_ref[...]
    sqgt = jnp.sum(gtb * gtb, axis=1, keepdims=True)
    sqgen = jnp.sum(genb * genb, axis=0, keepdims=True)
    lhs = (gtb + gtb).astype(jnp.bfloat16)
    rhs = genb.astype(jnp.bfloat16)
    cross2 = jax.lax.dot_general(
        lhs, rhs, (((1,), (0,)), ((), ())),
        preferred_element_type=jnp.float32)
    d2 = jnp.maximum((sqgt + sqgen) - cross2, 0.0)
    row_sum = jnp.sum(jnp.min(d2, axis=1)).reshape(1, 1)
    col_min = jnp.min(d2, axis=0)[None, :]

    @pl.when(i == 0)
    def _init():
        out_ref[...] = row_sum
        colmin_ref[...] = col_min

    @pl.when(i > 0)
    def _acc():
        out_ref[...] += row_sum
        colmin_ref[...] = jnp.minimum(colmin_ref[...], col_min)

    @pl.when(i == _NB - 1)
    def _fin():
        out_ref[...] = (out_ref[...] + jnp.sum(colmin_ref[...])) * (1.0 / _N)


def kernel(gt_points, gen_points):
    gen_t = gen_points.T

    out = pl.pallas_call(
        _chamfer_body,
        grid=(_NB,),
        in_specs=[
            pl.BlockSpec((_B, 3), lambda i: (i, 0)),
            pl.BlockSpec((3, _N), lambda i: (0, 0)),
        ],
        out_specs=pl.BlockSpec((1, 1), lambda i: (0, 0)),
        out_shape=jax.ShapeDtypeStruct((1, 1), jnp.float32),
        scratch_shapes=[pltpu.VMEM((1, _N), jnp.float32)],
        compiler_params=pltpu.CompilerParams(
            dimension_semantics=("arbitrary",),
        ),
    )(gt_points, gen_t)
    return out[0, 0]


# speedup vs baseline: 10.0112x; 1.2354x over previous
import jax
import jax.numpy as jnp
from jax.experimental import pallas as pl
from jax.experimental.pallas import tpu as pltpu

_N = 8192
_B = 1024
_NB = _N // _B


def _chamfer_body(gt_ref, gent_ref, out_ref, colmin_ref):
    i = pl.program_id(0)
    gtb = gt_ref[...]
    genb = gent_ref[...]
    sqgt = jnp.sum(gtb * gtb, axis=1, keepdims=True)
    sqgen = jnp.sum(genb * genb, axis=0, keepdims=True)
    lhs = (gtb + gtb).astype(jnp.bfloat16)
    rhs = genb.astype(jnp.bfloat16)
    cross2 = jax.lax.dot_general(
        lhs, rhs, (((1,), (0,)), ((), ())),
        preferred_element_type=jnp.float32)
    row_min = jnp.min(sqgen - cross2, axis=1) + sqgt[:, 0]
    row_sum = jnp.sum(jnp.maximum(row_min, 0.0)).reshape(1, 1)
    col_min = jnp.min(sqgt - cross2, axis=0)[None, :]

    @pl.when(i == 0)
    def _init():
        out_ref[...] = row_sum
        colmin_ref[...] = col_min

    @pl.when(i > 0)
    def _acc():
        out_ref[...] += row_sum
        colmin_ref[...] = jnp.minimum(colmin_ref[...], col_min)

    @pl.when(i == _NB - 1)
    def _fin():
        col_sum = jnp.sum(jnp.maximum(colmin_ref[...] + sqgen, 0.0))
        out_ref[...] = (out_ref[...] + col_sum) * (1.0 / _N)


def kernel(gt_points, gen_points):
    gen_t = gen_points.T

    out = pl.pallas_call(
        _chamfer_body,
        grid=(_NB,),
        in_specs=[
            pl.BlockSpec((_B, 3), lambda i: (i, 0)),
            pl.BlockSpec((3, _N), lambda i: (0, 0)),
        ],
        out_specs=pl.BlockSpec((1, 1), lambda i: (0, 0)),
        out_shape=jax.ShapeDtypeStruct((1, 1), jnp.float32),
        scratch_shapes=[pltpu.VMEM((1, _N), jnp.float32)],
        compiler_params=pltpu.CompilerParams(
            dimension_semantics=("arbitrary",),
        ),
    )(gt_points, gen_t)
    return out[0, 0]


# speedup vs baseline: 10.5564x; 1.0545x over previous
import jax
import jax.numpy as jnp
from jax.experimental import pallas as pl
from jax.experimental.pallas import tpu as pltpu

_N = 8192
_B = 2048
_NB = _N // _B


def _chamfer_body(gt_ref, gent_ref, out_ref, colmin_ref):
    i = pl.program_id(0)
    gtb = gt_ref[...]
    genb = gent_ref[...]
    sqgt = jnp.sum(gtb * gtb, axis=1, keepdims=True)
    sqgen = jnp.sum(genb * genb, axis=0, keepdims=True)
    sq_hi = sqgen.astype(jnp.bfloat16)
    sq_lo = (sqgen - sq_hi.astype(jnp.float32)).astype(jnp.bfloat16)
    lhs = jnp.concatenate(
        [(gtb + gtb).astype(jnp.bfloat16),
         jnp.full((_B, 2), -1.0, jnp.bfloat16)], axis=1)
    rhs = jnp.concatenate(
        [genb.astype(jnp.bfloat16), sq_hi, sq_lo], axis=0)
    m = jax.lax.dot_general(
        lhs, rhs, (((1,), (0,)), ((), ())),
        preferred_element_type=jnp.float32)
    row_min = sqgt[:, 0] - jnp.max(m, axis=1)
    row_sum = jnp.sum(jnp.maximum(row_min, 0.0)).reshape(1, 1)
    col_min = jnp.min(sqgt - m, axis=0)[None, :]

    @pl.when(i == 0)
    def _init():
        out_ref[...] = row_sum
        colmin_ref[...] = col_min

    @pl.when(i > 0)
    def _acc():
        out_ref[...] += row_sum
        colmin_ref[...] = jnp.minimum(colmin_ref[...], col_min)

    @pl.when(i == _NB - 1)
    def _fin():
        col_sum = jnp.sum(jnp.maximum(colmin_ref[...], 0.0))
        out_ref[...] = (out_ref[...] + col_sum) * (1.0 / _N)


def kernel(gt_points, gen_points):
    gen_t = gen_points.T

    out = pl.pallas_call(
        _chamfer_body,
        grid=(_NB,),
        in_specs=[
            pl.BlockSpec((_B, 3), lambda i: (i, 0)),
            pl.BlockSpec((3, _N), lambda i: (0, 0)),
        ],
        out_specs=pl.BlockSpec((1, 1), lambda i: (0, 0)),
        out_shape=jax.ShapeDtypeStruct((1, 1), jnp.float32),
        scratch_shapes=[pltpu.VMEM((1, _N), jnp.float32)],
        compiler_params=pltpu.CompilerParams(
            dimension_semantics=("arbitrary",),
        ),
    )(gt_points, gen_t)
    return out[0, 0]
